# baseline (speedup 1.0000x reference)
import jax
import jax.numpy as jnp
from jax import lax
from jax.experimental import pallas as pl
from jax.experimental.pallas import tpu as pltpu

_ROWS = 8192
_VOCAB = 16384
_BLOCK_ROWS = 256
_NBLK = _ROWS // _BLOCK_ROWS
_LANES = 128
_GROUPS = _VOCAB // _LANES


def _sloss_kernel(ts_ref, tv_ref, x_ref, o_ref, y_ref, acc_ref, cnt_ref):
    i = pl.program_id(0)

    @pl.when(i == 0)
    def _init():
        acc_ref[0] = 0.0
        cnt_ref[0] = 0.0

    x = x_ref[...]
    m = jnp.max(x, axis=-1, keepdims=True)
    s = jnp.sum(jnp.exp(x - m), axis=-1)
    lse = m[:, 0] + jnp.log(s)

    for r in range(_BLOCK_ROWS):
        t = ts_ref[0, i * _BLOCK_ROWS + r]
        off = pl.multiple_of((t >> 7) * _LANES, _LANES)
        y_ref[r, :] = x_ref[r, pl.ds(off, _LANES)]

    t = tv_ref[0, pl.ds(i * _BLOCK_ROWS, _BLOCK_ROWS)]
    lane = (t & (_LANES - 1))[:, None]
    iota = lax.broadcasted_iota(jnp.int32, (_BLOCK_ROWS, _LANES), 1)
    picked = jnp.sum(jnp.where(iota == lane, y_ref[...], 0.0), axis=-1)

    mask = t != 0
    acc_ref[0] += jnp.sum(jnp.where(mask, lse - picked, 0.0))
    cnt_ref[0] += jnp.sum(mask.astype(jnp.float32))

    @pl.when(i == _NBLK - 1)
    def _fin():
        o_ref[0] = acc_ref[0] / cnt_ref[0]


@jax.jit
def kernel(logits, targets):
    x = logits.reshape(_ROWS, _VOCAB)
    t = targets.reshape(1, _ROWS).astype(jnp.int32)

    out = pl.pallas_call(
        _sloss_kernel,
        grid=(_NBLK,),
        in_specs=[
            pl.BlockSpec(memory_space=pltpu.SMEM),
            pl.BlockSpec((1, _ROWS), lambda i: (0, 0)),
            pl.BlockSpec((_BLOCK_ROWS, _VOCAB), lambda i: (i, 0)),
        ],
        out_specs=pl.BlockSpec(memory_space=pltpu.SMEM),
        out_shape=jax.ShapeDtypeStruct((1,), jnp.float32),
        scratch_shapes=[
            pltpu.VMEM((_BLOCK_ROWS, _LANES), jnp.float32),
            pltpu.SMEM((1,), jnp.float32),
            pltpu.SMEM((1,), jnp.float32),
        ],
    )(t, t, x)
    return out[0]
